# baseline (speedup 1.0000x reference)
import functools

import jax
import jax.numpy as jnp
from jax import lax
from jax.experimental import pallas as pl
from jax.experimental.pallas import tpu as pltpu
from jax.experimental.pallas import tpu_sc as plsc

NUSER = 50000
NSHOW = 10000
NEDGE = 500000
FDIM = 128

NU_PAD = 50176
NS_PAD = 10240
GROUPS = 512
E_ROWS = GROUPS * 8
GCH = 64
E_PAD = GROUPS * 1024

NCORE = 2
NSUB = 16
NW = NCORE * NSUB

FLUSH = 128
PEND = 256 + 32
CAP = E_PAD + FLUSH
MAXBLK = CAP // FLUSH

_mesh = plsc.VectorSubcoreMesh(core_axis_name="c", subcore_axis_name="s")
_params = pltpu.CompilerParams(needs_layout_passes=False,
                               use_tc_tiling_on_sc=False)


def _make_seg(n_dst_pad, dw, compute_counts):
  rpt = n_dst_pad // NW
  nch = dw // 16

  def body(*refs):
    if compute_counts:
      (table, src3, dst3, zrow, zcnt,
       out, cnt_out, ls_out, ld_out, len_out,
       svm, dvm, ps, pd, rows, acc, cnt, cur_s, sem) = refs
    else:
      (table, src3, dst3, zrow, zcnt, cnt_in,
       out, svm, dvm, ps, pd, rows, acc, cnt, cur_s, sem) = refs
    cid = lax.axis_index("c")
    sid = lax.axis_index("s")
    gtid = sid * NCORE + cid
    base = gtid * rpt
    iota = lax.iota(jnp.int32, 16)
    m0 = iota < 1
    ones16f = jnp.full((16,), 1.0, jnp.float32)
    pltpu.sync_copy(zrow, acc)
    if compute_counts:
      pltpu.sync_copy(zcnt, cnt)
    else:
      pltpu.sync_copy(cnt_in.at[pl.ds(base, rpt)], cnt.at[pl.ds(0, rpt)])
    cur_s[0] = 0
    if compute_counts:
      cur_s[1] = 0

    def flush_one():
      g = pltpu.async_copy(table.at[ps.at[pl.ds(0, FLUSH)]], rows, sem)
      if compute_counts:
        wc = pl.multiple_of(cur_s[1], FLUSH)
        pltpu.sync_copy(ps.at[pl.ds(0, FLUSH)],
                        ls_out.at[gtid, pl.ds(wc, FLUSH)])
        pltpu.sync_copy(pd.at[pl.ds(0, FLUSH)],
                        ld_out.at[gtid, pl.ds(wc, FLUSH)])
        cur_s[1] = wc + FLUSH
      g.wait()

      def sub(sb, _):
        pdv = pd[pl.ds(sb * 16, 16)]
        for l in range(16):
          ldv = jnp.full((16,), pdv[l], jnp.int32)
          for c in range(nch):
            v = rows[sb * 16 + l, pl.ds(c * 16, 16)]
            plsc.addupdate_scatter(acc, (ldv, iota + c * 16), v)
          if compute_counts:
            plsc.addupdate_scatter(cnt, (ldv,), ones16f, mask=m0)
        return 0

      lax.fori_loop(0, FLUSH // 16, sub, 0)
      for t in range(9):
        ps[pl.ds(t * 16, 16)] = ps[pl.ds(FLUSH + t * 16, 16)]
        pd[pl.ds(t * 16, 16)] = pd[pl.ds(FLUSH + t * 16, 16)]
      cur_s[0] = cur_s[0] - FLUSH

    def row_body(rr, _):
      lanes = []
      for l in range(8):
        s = svm[rr, pl.ds(l * 16, 16)]
        d = dvm[rr, pl.ds(l * 16, 16)]
        m = (d >= base) & (d < base + rpt)
        ld = d - base
        inc = plsc.cumsum(m.astype(jnp.int32))
        lanes.append((s, ld, m, inc))
      off = cur_s[0]
      for s, ld, m, inc in lanes:
        pos = inc + (off - 1)
        plsc.store_scatter(ps, (pos,), s, mask=m)
        plsc.store_scatter(pd, (pos,), ld, mask=m)
        off = off + inc[15]
      cur_s[0] = off

      @pl.when(off >= FLUSH)
      def _():
        flush_one()
      return 0

    def chunk(ci, _):
      d1 = pltpu.async_copy(src3.at[pl.ds(ci * GCH, GCH)], svm, sem)
      d2 = pltpu.async_copy(dst3.at[pl.ds(ci * GCH, GCH)], dvm, sem)
      d1.wait()
      d2.wait()
      lax.fori_loop(0, GCH, row_body, 0)
      return 0

    lax.fori_loop(0, E_ROWS // GCH, chunk, 0)
    cur = cur_s[0]
    for t in range(8):
      ps[pl.ds(cur + t * 16, 16)] = iota + (gtid * 16) % 512
      pd[pl.ds(cur + t * 16, 16)] = jnp.full((16,), rpt, jnp.int32)
    flush_one()

    def norm(k, _):
      c16 = cnt[pl.ds(k * 16, 16)]
      r16 = 1.0 / jnp.maximum(c16, 1.0)
      for l in range(16):
        rv = jnp.full((16,), r16[l], jnp.float32)
        row = k * 16 + l
        for c in range(nch):
          acc[row, pl.ds(c * 16, 16)] = acc[row, pl.ds(c * 16, 16)] * rv
      return 0

    lax.fori_loop(0, rpt // 16, norm, 0)

    pltpu.sync_copy(acc.at[pl.ds(0, rpt)], out.at[pl.ds(base, rpt)])
    if compute_counts:
      pltpu.sync_copy(cnt.at[pl.ds(0, rpt)], cnt_out.at[pl.ds(base, rpt)])
      ps[pl.ds(0, 16)] = jnp.full((16,), cur_s[1], jnp.int32)
      pltpu.sync_copy(ps.at[pl.ds(0, 16)], len_out.at[gtid])

  outs = [jax.ShapeDtypeStruct((n_dst_pad, dw), jnp.float32)]
  if compute_counts:
    outs.append(jax.ShapeDtypeStruct((n_dst_pad,), jnp.float32))
    outs.append(jax.ShapeDtypeStruct((NW, CAP), jnp.int32))
    outs.append(jax.ShapeDtypeStruct((NW, CAP), jnp.int32))
    outs.append(jax.ShapeDtypeStruct((NW, 16), jnp.int32))
  return pl.kernel(
      body,
      out_type=outs if compute_counts else outs[0],
      mesh=_mesh,
      compiler_params=_params,
      scratch_types=[
          pltpu.VMEM((GCH, 128), jnp.int32),
          pltpu.VMEM((GCH, 128), jnp.int32),
          pltpu.VMEM((PEND,), jnp.int32),
          pltpu.VMEM((PEND,), jnp.int32),
          pltpu.VMEM((FLUSH, dw), jnp.float32),
          pltpu.VMEM((rpt + 8, dw), jnp.float32),
          pltpu.VMEM((rpt + 8,), jnp.float32),
          pltpu.SMEM((2,), jnp.int32),
          pltpu.SemaphoreType.DMA,
      ],
  )


def _make_replay(n_dst_pad, dw):
  rpt = n_dst_pad // NW
  nch = dw // 16

  def body(table, ls_in, ld_in, len_in, cnt_in, zrow,
           out, ps, pd, rows, acc, cnt, sem):
    cid = lax.axis_index("c")
    sid = lax.axis_index("s")
    gtid = sid * NCORE + cid
    base = gtid * rpt
    iota = lax.iota(jnp.int32, 16)
    pltpu.sync_copy(zrow, acc)
    pltpu.sync_copy(cnt_in.at[pl.ds(base, rpt)], cnt.at[pl.ds(0, rpt)])
    pltpu.sync_copy(len_in.at[gtid], ps.at[pl.ds(0, 16)])
    lenv = ps[pl.ds(0, 16)]
    nb = lenv[0] // FLUSH

    def blk(b, _):
      @pl.when(b < nb)
      def _():
        boff = pl.multiple_of(b * FLUSH, FLUSH)
        pltpu.sync_copy(ls_in.at[gtid, pl.ds(boff, FLUSH)],
                        ps.at[pl.ds(0, FLUSH)])
        pltpu.sync_copy(ld_in.at[gtid, pl.ds(boff, FLUSH)],
                        pd.at[pl.ds(0, FLUSH)])
        pltpu.async_copy(table.at[ps.at[pl.ds(0, FLUSH)]], rows, sem).wait()

        def sub(sb, _):
          pdv = pd[pl.ds(sb * 16, 16)]
          for l in range(16):
            ldv = jnp.full((16,), pdv[l], jnp.int32)
            for c in range(nch):
              v = rows[sb * 16 + l, pl.ds(c * 16, 16)]
              plsc.addupdate_scatter(acc, (ldv, iota + c * 16), v)
          return 0

        lax.fori_loop(0, FLUSH // 16, sub, 0)
      return 0

    lax.fori_loop(0, MAXBLK, blk, 0)

    def norm(k, _):
      c16 = cnt[pl.ds(k * 16, 16)]
      r16 = 1.0 / jnp.maximum(c16, 1.0)
      for l in range(16):
        rv = jnp.full((16,), r16[l], jnp.float32)
        row = k * 16 + l
        for c in range(nch):
          acc[row, pl.ds(c * 16, 16)] = acc[row, pl.ds(c * 16, 16)] * rv
      return 0

    lax.fori_loop(0, rpt // 16, norm, 0)
    pltpu.sync_copy(acc.at[pl.ds(0, rpt)], out.at[pl.ds(base, rpt)])

  return pl.kernel(
      body,
      out_type=jax.ShapeDtypeStruct((n_dst_pad, dw), jnp.float32),
      mesh=_mesh,
      compiler_params=_params,
      scratch_types=[
          pltpu.VMEM((PEND,), jnp.int32),
          pltpu.VMEM((PEND,), jnp.int32),
          pltpu.VMEM((FLUSH, dw), jnp.float32),
          pltpu.VMEM((rpt + 8, dw), jnp.float32),
          pltpu.VMEM((rpt + 8,), jnp.float32),
          pltpu.SemaphoreType.DMA,
      ],
  )


_seg_show_c = _make_seg(NS_PAD, FDIM, True)
_seg_user_c = _make_seg(NU_PAD, 64, True)
_rep_show = _make_replay(NS_PAD, FDIM)
_rep_user = _make_replay(NU_PAD, 64)


BLK = 1024


def _dense_show_kernel(agg_ref, xd_ref, wl_ref, wr_ref, b_ref,
                       out_ref, outh_ref=None, *, relu, emit_h):
  y = (jnp.dot(agg_ref[...], wl_ref[...], preferred_element_type=jnp.float32)
       + jnp.dot(xd_ref[...], wr_ref[...], preferred_element_type=jnp.float32)
       + b_ref[...])
  if relu:
    y = jnp.maximum(y, 0.0)
  out_ref[...] = y
  if emit_h:
    for h in range(2):
      outh_ref[h] = y[:, 64 * h:64 * h + 64]


def _dense_user_kernel(a0_ref, a1_ref, xd_ref, wl_ref, wr_ref, b_ref,
                       out_ref, *, relu):
  agg = jnp.concatenate([a0_ref[...], a1_ref[...]], axis=-1)
  y = (jnp.dot(agg, wl_ref[...], preferred_element_type=jnp.float32)
       + jnp.dot(xd_ref[...], wr_ref[...], preferred_element_type=jnp.float32)
       + b_ref[...])
  if relu:
    y = jnp.maximum(y, 0.0)
  out_ref[...] = y


def _dense_show(agg, xd, wl, wr, b, relu, emit_h):
  n = agg.shape[0]
  grid = (n // BLK,)
  out_shapes = [jax.ShapeDtypeStruct((n, FDIM), jnp.float32)]
  out_specs = [pl.BlockSpec((BLK, FDIM), lambda i: (i, 0))]
  if emit_h:
    out_shapes.append(jax.ShapeDtypeStruct((2, n, 64), jnp.float32))
    out_specs.append(pl.BlockSpec((2, BLK, 64), lambda i: (0, i, 0)))
  return pl.pallas_call(
      functools.partial(_dense_show_kernel, relu=relu, emit_h=emit_h),
      grid=grid,
      in_specs=[
          pl.BlockSpec((BLK, FDIM), lambda i: (i, 0)),
          pl.BlockSpec((BLK, FDIM), lambda i: (i, 0)),
          pl.BlockSpec((FDIM, FDIM), lambda i: (0, 0)),
          pl.BlockSpec((FDIM, FDIM), lambda i: (0, 0)),
          pl.BlockSpec((1, FDIM), lambda i: (0, 0)),
      ],
      out_specs=out_specs if emit_h else out_specs[0],
      out_shape=out_shapes if emit_h else out_shapes[0],
  )(agg, xd, wl, wr, b)


def _dense_user(a0, a1, xd, wl, wr, b, relu):
  n = a0.shape[0]
  grid = (n // BLK,)
  return pl.pallas_call(
      functools.partial(_dense_user_kernel, relu=relu),
      grid=grid,
      in_specs=[
          pl.BlockSpec((BLK, 64), lambda i: (i, 0)),
          pl.BlockSpec((BLK, 64), lambda i: (i, 0)),
          pl.BlockSpec((BLK, FDIM), lambda i: (i, 0)),
          pl.BlockSpec((FDIM, FDIM), lambda i: (0, 0)),
          pl.BlockSpec((FDIM, FDIM), lambda i: (0, 0)),
          pl.BlockSpec((1, FDIM), lambda i: (0, 0)),
      ],
      out_specs=pl.BlockSpec((BLK, FDIM), lambda i: (i, 0)),
      out_shape=jax.ShapeDtypeStruct((n, FDIM), jnp.float32),
  )(a0, a1, xd, wl, wr, b)


def _prep_edges(ei, n_src, n_dst):
  src = ei[0].astype(jnp.int32)
  dst = ei[1].astype(jnp.int32)
  npad = E_PAD - NEDGE
  fill = jnp.arange(npad, dtype=jnp.int32)
  psrc = (fill * 97) % n_src
  pdst = n_dst + (fill % 128)
  src2 = jnp.concatenate([src, psrc]).reshape(E_ROWS, 128)
  dst2 = jnp.concatenate([dst, pdst]).reshape(E_ROWS, 128)
  return src2, dst2


def _halves(x):
  n = x.shape[0]
  return jnp.transpose(x.reshape(n, 2, 64), (1, 0, 2))


@jax.jit
def kernel(x_user, x_show, edge_index_attended, edge_index_rev_attended,
           W1_att_l, b1_att, W1_att_r, W1_rev_l, b1_rev, W1_rev_r,
           W2_att_l, b2_att, W2_att_r, W2_rev_l, b2_rev, W2_rev_r):
  f32 = jnp.float32
  src_att, dst_att = _prep_edges(edge_index_attended, NUSER, NSHOW)
  src_rev, dst_rev = _prep_edges(edge_index_rev_attended, NSHOW, NUSER)

  rpt_s = NS_PAD // NW
  rpt_u = NU_PAD // NW
  zrow_s = jnp.zeros((rpt_s + 8, FDIM), f32)
  zcnt_s = jnp.zeros((rpt_s + 8,), f32)
  zrow_u = jnp.zeros((rpt_u + 8, 64), f32)
  zcnt_u = jnp.zeros((rpt_u + 8,), f32)

  xu_pad = jnp.pad(x_user, ((0, NU_PAD - NUSER), (0, 0)))
  xs_pad = jnp.pad(x_show, ((0, NS_PAD - NSHOW), (0, 0)))
  xs_h = _halves(x_show)

  agg1_s, cnt_show, lsa, lda, lena = _seg_show_c(
      x_user, src_att, dst_att, zrow_s, zcnt_s)
  agg1_u0, cnt_user, lsr, ldr, lenr = _seg_user_c(
      xs_h[0], src_rev, dst_rev, zrow_u, zcnt_u)
  agg1_u1 = _rep_user(xs_h[1], lsr, ldr, lenr, cnt_user, zrow_u)

  h_show, h_show_h = _dense_show(agg1_s, xs_pad, W1_att_l, W1_att_r,
                                 b1_att.reshape(1, FDIM), relu=True,
                                 emit_h=True)
  h_user = _dense_user(agg1_u0, agg1_u1, xu_pad, W1_rev_l, W1_rev_r,
                       b1_rev.reshape(1, FDIM), relu=True)

  agg2_s = _rep_show(h_user, lsa, lda, lena, cnt_show, zrow_s)
  agg2_u0 = _rep_user(h_show_h[0], lsr, ldr, lenr, cnt_user, zrow_u)
  agg2_u1 = _rep_user(h_show_h[1], lsr, ldr, lenr, cnt_user, zrow_u)

  out_show = _dense_show(agg2_s, h_show, W2_att_l, W2_att_r,
                         b2_att.reshape(1, FDIM), relu=False, emit_h=False)
  out_user = _dense_user(agg2_u0, agg2_u1, h_user, W2_rev_l, W2_rev_r,
                         b2_rev.reshape(1, FDIM), relu=False)
  return (out_user[:NUSER], out_show[:NSHOW])


# speedup vs baseline: 1.0232x; 1.0232x over previous
import functools

import jax
import jax.numpy as jnp
from jax import lax
from jax.experimental import pallas as pl
from jax.experimental.pallas import tpu as pltpu
from jax.experimental.pallas import tpu_sc as plsc

NUSER = 50000
NSHOW = 10000
NEDGE = 500000
FDIM = 128

NU_PAD = 50176
NS_PAD = 10240
GROUPS = 512
E_ROWS = GROUPS * 8
GCH = 64
E_PAD = GROUPS * 1024

NCORE = 2
NSUB = 16
NW = NCORE * NSUB

FLUSH = 128
PEND = 256 + 32

_mesh = plsc.VectorSubcoreMesh(core_axis_name="c", subcore_axis_name="s")
_params = pltpu.CompilerParams(needs_layout_passes=False,
                               use_tc_tiling_on_sc=False)


def _make_seg(n_dst_pad, dw, compute_counts):
  rpt = n_dst_pad // NW
  nch = dw // 16

  def body(*refs):
    if compute_counts:
      (table, src3, dst3, zrow, zcnt,
       out, cnt_out, svm, dvm, ps, pd, rows, acc, cnt, cur_s, sem) = refs
    else:
      (table, src3, dst3, zrow, zcnt, cnt_in,
       out, svm, dvm, ps, pd, rows, acc, cnt, cur_s, sem) = refs
    cid = lax.axis_index("c")
    sid = lax.axis_index("s")
    gtid = sid * NCORE + cid
    base = gtid * rpt
    iota = lax.iota(jnp.int32, 16)
    m0 = iota < 1
    ones16f = jnp.full((16,), 1.0, jnp.float32)
    pltpu.sync_copy(zrow, acc)
    if compute_counts:
      pltpu.sync_copy(zcnt, cnt)
    else:
      pltpu.sync_copy(cnt_in.at[pl.ds(base, rpt)], cnt.at[pl.ds(0, rpt)])
    cur_s[0] = 0

    def flush_one():
      pltpu.async_copy(table.at[ps.at[pl.ds(0, FLUSH)]], rows, sem).wait()

      def sub(sb, _):
        pdv = pd[pl.ds(sb * 16, 16)]
        for l in range(16):
          ldv = jnp.full((16,), pdv[l], jnp.int32)
          for c in range(nch):
            v = rows[sb * 16 + l, pl.ds(c * 16, 16)]
            plsc.addupdate_scatter(acc, (ldv, iota + c * 16), v)
          if compute_counts:
            plsc.addupdate_scatter(cnt, (ldv,), ones16f, mask=m0)
        return 0

      lax.fori_loop(0, FLUSH // 16, sub, 0)
      for t in range(9):
        ps[pl.ds(t * 16, 16)] = ps[pl.ds(FLUSH + t * 16, 16)]
        pd[pl.ds(t * 16, 16)] = pd[pl.ds(FLUSH + t * 16, 16)]
      cur_s[0] = cur_s[0] - FLUSH

    def row_body(rr, _):
      lanes = []
      for l in range(8):
        s = svm[rr, pl.ds(l * 16, 16)]
        d = dvm[rr, pl.ds(l * 16, 16)]
        m = (d >= base) & (d < base + rpt)
        ld = d - base
        inc = plsc.cumsum(m.astype(jnp.int32))
        lanes.append((s, ld, m, inc))
      off = cur_s[0]
      for s, ld, m, inc in lanes:
        pos = inc + (off - 1)
        plsc.store_scatter(ps, (pos,), s, mask=m)
        plsc.store_scatter(pd, (pos,), ld, mask=m)
        off = off + inc[15]
      cur_s[0] = off

      @pl.when(off >= FLUSH)
      def _():
        flush_one()
      return 0

    def chunk(ci, _):
      d1 = pltpu.async_copy(src3.at[pl.ds(ci * GCH, GCH)], svm, sem)
      d2 = pltpu.async_copy(dst3.at[pl.ds(ci * GCH, GCH)], dvm, sem)
      d1.wait()
      d2.wait()
      lax.fori_loop(0, GCH, row_body, 0)
      return 0

    lax.fori_loop(0, E_ROWS // GCH, chunk, 0)
    cur = cur_s[0]
    for t in range(8):
      ps[pl.ds(cur + t * 16, 16)] = iota + (gtid * 16) % 512
      pd[pl.ds(cur + t * 16, 16)] = jnp.full((16,), rpt, jnp.int32)
    flush_one()

    def norm(k, _):
      c16 = cnt[pl.ds(k * 16, 16)]
      r16 = 1.0 / jnp.maximum(c16, 1.0)
      for l in range(16):
        rv = jnp.full((16,), r16[l], jnp.float32)
        row = k * 16 + l
        for c in range(nch):
          acc[row, pl.ds(c * 16, 16)] = acc[row, pl.ds(c * 16, 16)] * rv
      return 0

    lax.fori_loop(0, rpt // 16, norm, 0)

    pltpu.sync_copy(acc.at[pl.ds(0, rpt)], out.at[pl.ds(base, rpt)])
    if compute_counts:
      pltpu.sync_copy(cnt.at[pl.ds(0, rpt)], cnt_out.at[pl.ds(base, rpt)])

  outs = [jax.ShapeDtypeStruct((n_dst_pad, dw), jnp.float32)]
  if compute_counts:
    outs.append(jax.ShapeDtypeStruct((n_dst_pad,), jnp.float32))
  return pl.kernel(
      body,
      out_type=outs if compute_counts else outs[0],
      mesh=_mesh,
      compiler_params=_params,
      scratch_types=[
          pltpu.VMEM((GCH, 128), jnp.int32),
          pltpu.VMEM((GCH, 128), jnp.int32),
          pltpu.VMEM((PEND,), jnp.int32),
          pltpu.VMEM((PEND,), jnp.int32),
          pltpu.VMEM((FLUSH, dw), jnp.float32),
          pltpu.VMEM((rpt + 8, dw), jnp.float32),
          pltpu.VMEM((rpt + 8,), jnp.float32),
          pltpu.SMEM((1,), jnp.int32),
          pltpu.SemaphoreType.DMA,
      ],
  )


_seg_show_c = _make_seg(NS_PAD, FDIM, True)
_seg_show_u = _make_seg(NS_PAD, FDIM, False)
_seg_user_c = _make_seg(NU_PAD, 64, True)
_seg_user_u = _make_seg(NU_PAD, 64, False)


BLK = 1024


def _dense_show_kernel(agg_ref, xd_ref, wl_ref, wr_ref, b_ref,
                       out_ref, outh_ref=None, *, relu, emit_h):
  y = (jnp.dot(agg_ref[...], wl_ref[...], preferred_element_type=jnp.float32)
       + jnp.dot(xd_ref[...], wr_ref[...], preferred_element_type=jnp.float32)
       + b_ref[...])
  if relu:
    y = jnp.maximum(y, 0.0)
  out_ref[...] = y
  if emit_h:
    for h in range(2):
      outh_ref[h] = y[:, 64 * h:64 * h + 64]


def _dense_user_kernel(a0_ref, a1_ref, xd_ref, wl_ref, wr_ref, b_ref,
                       out_ref, *, relu):
  agg = jnp.concatenate([a0_ref[...], a1_ref[...]], axis=-1)
  y = (jnp.dot(agg, wl_ref[...], preferred_element_type=jnp.float32)
       + jnp.dot(xd_ref[...], wr_ref[...], preferred_element_type=jnp.float32)
       + b_ref[...])
  if relu:
    y = jnp.maximum(y, 0.0)
  out_ref[...] = y


def _dense_show(agg, xd, wl, wr, b, relu, emit_h):
  n = agg.shape[0]
  grid = (n // BLK,)
  out_shapes = [jax.ShapeDtypeStruct((n, FDIM), jnp.float32)]
  out_specs = [pl.BlockSpec((BLK, FDIM), lambda i: (i, 0))]
  if emit_h:
    out_shapes.append(jax.ShapeDtypeStruct((2, n, 64), jnp.float32))
    out_specs.append(pl.BlockSpec((2, BLK, 64), lambda i: (0, i, 0)))
  return pl.pallas_call(
      functools.partial(_dense_show_kernel, relu=relu, emit_h=emit_h),
      grid=grid,
      in_specs=[
          pl.BlockSpec((BLK, FDIM), lambda i: (i, 0)),
          pl.BlockSpec((BLK, FDIM), lambda i: (i, 0)),
          pl.BlockSpec((FDIM, FDIM), lambda i: (0, 0)),
          pl.BlockSpec((FDIM, FDIM), lambda i: (0, 0)),
          pl.BlockSpec((1, FDIM), lambda i: (0, 0)),
      ],
      out_specs=out_specs if emit_h else out_specs[0],
      out_shape=out_shapes if emit_h else out_shapes[0],
  )(agg, xd, wl, wr, b)


def _dense_user(a0, a1, xd, wl, wr, b, relu):
  n = a0.shape[0]
  grid = (n // BLK,)
  return pl.pallas_call(
      functools.partial(_dense_user_kernel, relu=relu),
      grid=grid,
      in_specs=[
          pl.BlockSpec((BLK, 64), lambda i: (i, 0)),
          pl.BlockSpec((BLK, 64), lambda i: (i, 0)),
          pl.BlockSpec((BLK, FDIM), lambda i: (i, 0)),
          pl.BlockSpec((FDIM, FDIM), lambda i: (0, 0)),
          pl.BlockSpec((FDIM, FDIM), lambda i: (0, 0)),
          pl.BlockSpec((1, FDIM), lambda i: (0, 0)),
      ],
      out_specs=pl.BlockSpec((BLK, FDIM), lambda i: (i, 0)),
      out_shape=jax.ShapeDtypeStruct((n, FDIM), jnp.float32),
  )(a0, a1, xd, wl, wr, b)


def _prep_edges(ei, n_src, n_dst):
  src = ei[0].astype(jnp.int32)
  dst = ei[1].astype(jnp.int32)
  npad = E_PAD - NEDGE
  fill = jnp.arange(npad, dtype=jnp.int32)
  psrc = (fill * 97) % n_src
  pdst = n_dst + (fill % 128)
  src2 = jnp.concatenate([src, psrc]).reshape(E_ROWS, 128)
  dst2 = jnp.concatenate([dst, pdst]).reshape(E_ROWS, 128)
  return src2, dst2


def _halves(x):
  n = x.shape[0]
  return jnp.transpose(x.reshape(n, 2, 64), (1, 0, 2))


@jax.jit
def kernel(x_user, x_show, edge_index_attended, edge_index_rev_attended,
           W1_att_l, b1_att, W1_att_r, W1_rev_l, b1_rev, W1_rev_r,
           W2_att_l, b2_att, W2_att_r, W2_rev_l, b2_rev, W2_rev_r):
  f32 = jnp.float32
  src_att, dst_att = _prep_edges(edge_index_attended, NUSER, NSHOW)
  src_rev, dst_rev = _prep_edges(edge_index_rev_attended, NSHOW, NUSER)

  rpt_s = NS_PAD // NW
  rpt_u = NU_PAD // NW
  zrow_s = jnp.zeros((rpt_s + 8, FDIM), f32)
  zcnt_s = jnp.zeros((rpt_s + 8,), f32)
  zrow_u = jnp.zeros((rpt_u + 8, 64), f32)
  zcnt_u = jnp.zeros((rpt_u + 8,), f32)

  xu_pad = jnp.pad(x_user, ((0, NU_PAD - NUSER), (0, 0)))
  xs_pad = jnp.pad(x_show, ((0, NS_PAD - NSHOW), (0, 0)))
  xs_h = _halves(x_show)

  agg1_s, cnt_show = _seg_show_c(x_user, src_att, dst_att, zrow_s, zcnt_s)
  agg1_u0, cnt_user = _seg_user_c(xs_h[0], src_rev, dst_rev, zrow_u, zcnt_u)
  agg1_u1 = _seg_user_u(xs_h[1], src_rev, dst_rev, zrow_u, zcnt_u, cnt_user)

  h_show, h_show_h = _dense_show(agg1_s, xs_pad, W1_att_l, W1_att_r,
                                 b1_att.reshape(1, FDIM), relu=True,
                                 emit_h=True)
  h_user = _dense_user(agg1_u0, agg1_u1, xu_pad, W1_rev_l, W1_rev_r,
                       b1_rev.reshape(1, FDIM), relu=True)

  agg2_s = _seg_show_u(h_user, src_att, dst_att, zrow_s, zcnt_s, cnt_show)
  agg2_u0 = _seg_user_u(h_show_h[0], src_rev, dst_rev, zrow_u, zcnt_u,
                        cnt_user)
  agg2_u1 = _seg_user_u(h_show_h[1], src_rev, dst_rev, zrow_u, zcnt_u,
                        cnt_user)

  out_show = _dense_show(agg2_s, h_show, W2_att_l, W2_att_r,
                         b2_att.reshape(1, FDIM), relu=False, emit_h=False)
  out_user = _dense_user(agg2_u0, agg2_u1, h_user, W2_rev_l, W2_rev_r,
                         b2_rev.reshape(1, FDIM), relu=False)
  return (out_user[:NUSER], out_show[:NSHOW])
